# Initial kernel scaffold; baseline (speedup 1.0000x reference)
import jax
import jax.numpy as jnp
from jax.experimental import pallas as pl


def kernel(x, node_type, W, b):
    raise NotImplementedError("write your pallas kernel here")


# speedup vs baseline: 1.8495x; 1.8495x over previous
import functools

import jax
import jax.numpy as jnp
from jax import lax
from jax.experimental import pallas as pl
from jax.experimental.pallas import tpu as pltpu
from jax.experimental.pallas import tpu_sc as plsc

N = 4096
D_IN = 1024
D_OUT = 1024
T = 8
BLK = 256
NPAD = N + T * BLK
NB = NPAD // BLK
NBT = 128

NC = 2
NS = 16
NW = NC * NS
ROWS_W = N // NW
CHUNK = 32
NCH = ROWS_W // CHUNK


def _cumsum(x, axis):
    n = x.shape[axis]
    s = 1
    while s < n:
        zshape = list(x.shape)
        zshape[axis] = s
        zeros = jnp.zeros(zshape, x.dtype)
        shifted = jnp.concatenate(
            [zeros, lax.slice_in_dim(x, 0, n - s, axis=axis)], axis=axis)
        x = x + shifted
        s *= 2
    return x


def _route_body(nt_ref, dst_ref, bt_ref):
    nt = nt_ref[...]
    dst = jnp.zeros(nt.shape, jnp.int32)
    j_iota = lax.broadcasted_iota(jnp.int32, (1, NBT), 1)
    bt = jnp.zeros((1, NBT), jnp.int32)
    p = jnp.int32(0)
    for t in range(T):
        if t > 0:
            bt = bt + (j_iota >= p // BLK).astype(jnp.int32)
        m = nt == t
        mi = m.astype(jnp.int32)
        inc = _cumsum(mi, axis=1)
        rowtot = inc[:, -1:]
        rowoff = _cumsum(rowtot, axis=0) - rowtot
        rank = inc - mi + rowoff
        dst = jnp.where(m, p + rank, dst)
        cnt = jnp.sum(mi)
        p = p + ((cnt + BLK - 1) // BLK) * BLK
    dst_ref[...] = dst
    bt_ref[...] = jnp.where(j_iota < p // BLK, bt, -1)


def _mm_body(bt_ref, xs_ref, w_ref, b_ref, out_ref):
    @pl.when(bt_ref[pl.program_id(0)] >= 0)
    def _():
        out_ref[...] = (
            jnp.dot(xs_ref[...], w_ref[0], preferred_element_type=jnp.float32)
            + b_ref[0])


def _sc_mesh():
    return plsc.VectorSubcoreMesh(
        core_axis_name="c", subcore_axis_name="s", num_cores=NC)


def _sc_scatter_rows(x, dst3):
    @functools.partial(
        pl.kernel,
        mesh=_sc_mesh(),
        out_type=jax.ShapeDtypeStruct((NPAD, D_IN), jnp.float32),
        scratch_types=[
            pltpu.VMEM((NCH, CHUNK), jnp.int32),
            pltpu.VMEM((CHUNK, D_IN), jnp.float32),
            pltpu.SemaphoreType.DMA,
        ],
    )
    def k(x_hbm, dst_hbm, xs_hbm, idx_v, buf, sem):
        wid = lax.axis_index("s") * NC + lax.axis_index("c")
        base = wid * ROWS_W
        pltpu.sync_copy(dst_hbm.at[wid], idx_v)
        for ch in range(NCH):
            pltpu.sync_copy(x_hbm.at[pl.ds(base + ch * CHUNK, CHUNK)], buf)
            pltpu.async_copy(buf, xs_hbm.at[idx_v.at[ch]], sem).wait()

    return k(x, dst3)


def _sc_gather_rows(ys, dst3):
    @functools.partial(
        pl.kernel,
        mesh=_sc_mesh(),
        out_type=jax.ShapeDtypeStruct((N, D_OUT), jnp.float32),
        scratch_types=[
            pltpu.VMEM((NCH, CHUNK), jnp.int32),
            pltpu.VMEM((CHUNK, D_OUT), jnp.float32),
            pltpu.SemaphoreType.DMA,
        ],
    )
    def k(ys_hbm, dst_hbm, out_hbm, idx_v, buf, sem):
        wid = lax.axis_index("s") * NC + lax.axis_index("c")
        base = wid * ROWS_W
        pltpu.sync_copy(dst_hbm.at[wid], idx_v)
        for ch in range(NCH):
            pltpu.async_copy(ys_hbm.at[idx_v.at[ch]], buf, sem).wait()
            pltpu.sync_copy(buf, out_hbm.at[pl.ds(base + ch * CHUNK, CHUNK)])

    return k(ys, dst3)


def _mm_grid_spec():
    return pltpu.PrefetchScalarGridSpec(
        num_scalar_prefetch=1,
        grid=(NB,),
        in_specs=[
            pl.BlockSpec((BLK, D_IN), lambda j, bt: (j, 0)),
            pl.BlockSpec((1, D_IN, D_OUT),
                         lambda j, bt: (jnp.maximum(bt[j], 0), 0, 0)),
            pl.BlockSpec((1, 1, D_OUT),
                         lambda j, bt: (jnp.maximum(bt[j], 0), 0, 0)),
        ],
        out_specs=pl.BlockSpec((BLK, D_OUT), lambda j, bt: (j, 0)),
    )


def kernel(x, node_type, W, b):
    nt2d = node_type.astype(jnp.int32).reshape(T, N // T)
    dst2d, btv = pl.pallas_call(
        _route_body,
        out_shape=(jax.ShapeDtypeStruct((T, N // T), jnp.int32),
                   jax.ShapeDtypeStruct((1, NBT), jnp.int32)),
    )(nt2d)
    dst3 = dst2d.reshape(NW, NCH, CHUNK)
    xs = _sc_scatter_rows(x, dst3)
    ys = pl.pallas_call(
        _mm_body,
        grid_spec=_mm_grid_spec(),
        out_shape=jax.ShapeDtypeStruct((NPAD, D_OUT), jnp.float32),
    )(btv.reshape(NBT), xs, W, b.reshape(T, 1, D_OUT))
    return _sc_gather_rows(ys, dst3)


# speedup vs baseline: 2.1860x; 1.1819x over previous
import functools

import jax
import jax.numpy as jnp
from jax import lax
from jax.experimental import pallas as pl
from jax.experimental.pallas import tpu as pltpu
from jax.experimental.pallas import tpu_sc as plsc

N = 4096
D_IN = 1024
D_OUT = 1024
T = 8
BLK = 512
NB = N // BLK
NBT = 128

NC = 2
NS = 16
NW = NC * NS
ROWS_W = N // NW
CHUNK = 64
NCH = ROWS_W // CHUNK
NTC = N // T


def _cumsum(x, axis):
    n = x.shape[axis]
    s = 1
    while s < n:
        zshape = list(x.shape)
        zshape[axis] = s
        zeros = jnp.zeros(zshape, x.dtype)
        shifted = jnp.concatenate(
            [zeros, lax.slice_in_dim(x, 0, n - s, axis=axis)], axis=axis)
        x = x + shifted
        s *= 2
    return x


def _route_body(nt_ref, dst_ref, off_ref):
    nt = nt_ref[...]
    dst = jnp.zeros(nt.shape, jnp.int32)
    j_iota = lax.broadcasted_iota(jnp.int32, (1, NBT), 1)
    off = jnp.full((1, NBT), N, jnp.int32)
    p = jnp.int32(0)
    for t in range(T):
        off = jnp.where(j_iota == t, p, off)
        m = nt == t
        mi = m.astype(jnp.int32)
        inc = _cumsum(mi, axis=1)
        rowtot = inc[:, -1:]
        rowoff = _cumsum(rowtot, axis=0) - rowtot
        rank = inc - mi + rowoff
        dst = jnp.where(m, p + rank, dst)
        p = p + jnp.sum(mi)
    dst_ref[...] = dst
    off_ref[...] = off


def _mm_body(off_ref, xs_ref, w_hbm, b_ref, out_ref, w_v, sems):
    j = pl.program_id(0)
    base = j * BLK

    @pl.when(j == 0)
    def _():
        for t in range(T):
            @pl.when(off_ref[0, t + 1] > off_ref[0, t])
            def _(t=t):
                pltpu.make_async_copy(
                    w_hbm.at[t], w_v.at[t], sems.at[t]).start()

    rowids = base + lax.broadcasted_iota(jnp.int32, (BLK, 1), 0)
    for t in range(T):
        lo_t = off_ref[0, t]
        hi_t = off_ref[0, t + 1]

        @pl.when((hi_t > lo_t) & (hi_t > base) & (lo_t < base + BLK))
        def _(t=t, lo_t=lo_t, hi_t=hi_t):
            @pl.when(j == lo_t // BLK)
            def _():
                pltpu.make_async_copy(
                    w_hbm.at[t], w_v.at[t], sems.at[t]).wait()

            mask = (rowids >= lo_t) & (rowids < hi_t)
            y = (jnp.dot(xs_ref[...], w_v[t],
                         preferred_element_type=jnp.float32)
                 + b_ref[t])
            out_ref[...] = jnp.where(mask, y, out_ref[...])


def _sc_mesh():
    return plsc.VectorSubcoreMesh(
        core_axis_name="c", subcore_axis_name="s", num_cores=NC)


def _load_worker_idx(dst_hbm, idx_v, wid):
    per_row = NTC // ROWS_W
    r = wid // per_row
    col = (wid % per_row) * ROWS_W
    pltpu.sync_copy(dst_hbm.at[r, pl.ds(col, ROWS_W)], idx_v)


def _sc_scatter_rows(x, dst2d):
    @functools.partial(
        pl.kernel,
        mesh=_sc_mesh(),
        out_type=jax.ShapeDtypeStruct((N, D_IN), jnp.float32),
        scratch_types=[
            pltpu.VMEM((ROWS_W,), jnp.int32),
            pltpu.VMEM((CHUNK,), jnp.int32),
            pltpu.VMEM((CHUNK, D_IN), jnp.float32),
            pltpu.SemaphoreType.DMA,
        ],
    )
    def k(x_hbm, dst_hbm, xs_hbm, idx_v, idx_c, buf, sem):
        wid = lax.axis_index("s") * NC + lax.axis_index("c")
        base = wid * ROWS_W
        _load_worker_idx(dst_hbm, idx_v, wid)
        for ch in range(NCH):
            for v in range(CHUNK // 16):
                idx_c[pl.ds(v * 16, 16)] = idx_v[pl.ds(ch * CHUNK + v * 16, 16)]
            pltpu.sync_copy(x_hbm.at[pl.ds(base + ch * CHUNK, CHUNK)], buf)
            pltpu.async_copy(buf, xs_hbm.at[idx_c], sem).wait()

    return k(x, dst2d)


def _sc_gather_rows(ys, dst2d):
    @functools.partial(
        pl.kernel,
        mesh=_sc_mesh(),
        out_type=jax.ShapeDtypeStruct((N, D_OUT), jnp.float32),
        scratch_types=[
            pltpu.VMEM((ROWS_W,), jnp.int32),
            pltpu.VMEM((CHUNK,), jnp.int32),
            pltpu.VMEM((CHUNK, D_OUT), jnp.float32),
            pltpu.SemaphoreType.DMA,
        ],
    )
    def k(ys_hbm, dst_hbm, out_hbm, idx_v, idx_c, buf, sem):
        wid = lax.axis_index("s") * NC + lax.axis_index("c")
        base = wid * ROWS_W
        _load_worker_idx(dst_hbm, idx_v, wid)
        for ch in range(NCH):
            for v in range(CHUNK // 16):
                idx_c[pl.ds(v * 16, 16)] = idx_v[pl.ds(ch * CHUNK + v * 16, 16)]
            pltpu.async_copy(ys_hbm.at[idx_c], buf, sem).wait()
            pltpu.sync_copy(buf, out_hbm.at[pl.ds(base + ch * CHUNK, CHUNK)])

    return k(ys, dst2d)


def _mm_grid_spec():
    return pltpu.PrefetchScalarGridSpec(
        num_scalar_prefetch=1,
        grid=(NB,),
        in_specs=[
            pl.BlockSpec((BLK, D_IN), lambda j, off: (j, 0)),
            pl.BlockSpec(memory_space=pltpu.MemorySpace.HBM),
            pl.BlockSpec(memory_space=pltpu.MemorySpace.VMEM),
        ],
        out_specs=pl.BlockSpec((BLK, D_OUT), lambda j, off: (j, 0)),
        scratch_shapes=[
            pltpu.VMEM((T, D_IN, D_OUT), jnp.float32),
            pltpu.SemaphoreType.DMA((T,)),
        ],
    )


def kernel(x, node_type, W, b):
    nt2d = node_type.astype(jnp.int32).reshape(T, NTC)
    dst2d, off = pl.pallas_call(
        _route_body,
        out_shape=(jax.ShapeDtypeStruct((T, NTC), jnp.int32),
                   jax.ShapeDtypeStruct((1, NBT), jnp.int32)),
    )(nt2d)
    xs = _sc_scatter_rows(x, dst2d)
    ys = pl.pallas_call(
        _mm_body,
        grid_spec=_mm_grid_spec(),
        out_shape=jax.ShapeDtypeStruct((N, D_OUT), jnp.float32),
    )(off, xs, W, b.reshape(T, 1, D_OUT))
    return _sc_gather_rows(ys, dst2d)
